# baseline (speedup 1.0000x reference)
import jax
import jax.numpy as jnp
from jax import lax
from jax.experimental import pallas as pl
from jax.experimental.pallas import tpu as pltpu
from jax.experimental.pallas import tpu_sc as plsc

_BS = 8


_IDX_W = 112


def _sc_lookup_body(table_hbm, rows_hbm, idx_v, rows_v, sem):
    first = (lax.axis_index("c") == 0) & (lax.axis_index("s") == 0)

    @pl.when(first)
    def _():
        S = rows_hbm.shape[0]
        for j in range(2):
            for i in range(_IDX_W // 16):
                idx_v[j, pl.ds(i * 16, 16)] = lax.iota(jnp.int32, 16) + (
                    j * _IDX_W + i * 16
                )
        burst = [
            pltpu.async_copy(
                table_hbm.at[idx_v.at[j]],
                rows_v.at[pl.ds(j * _IDX_W, _IDX_W)],
                sem,
            )
            for j in range(2)
        ]
        for c in burst:
            c.wait()
        pltpu.sync_copy(rows_v.at[pl.ds(0, S)], rows_hbm)


def _tc_splat_body(rows_ref, out_ref):
    out_ref[...] = jnp.broadcast_to(
        rows_ref[...][:, :, None], out_ref.shape
    )


def kernel(x, pos_table):
    N, S = x.shape
    D = pos_table.shape[1]

    mesh = plsc.VectorSubcoreMesh(core_axis_name="c", subcore_axis_name="s")
    sc_lookup = pl.kernel(
        _sc_lookup_body,
        out_type=jax.ShapeDtypeStruct((S, D), jnp.float32),
        mesh=mesh,
        scratch_types=[
            pltpu.VMEM((2, _IDX_W), jnp.int32),
            pltpu.VMEM((2 * _IDX_W, D), jnp.float32),
            pltpu.SemaphoreType.DMA,
        ],
        compiler_params=pltpu.CompilerParams(use_tc_tiling_on_sc=False),
    )
    rows = sc_lookup(pos_table)

    bs = _BS
    while S % bs:
        bs //= 2
    out_t = pl.pallas_call(
        _tc_splat_body,
        grid=(S // bs,),
        in_specs=[pl.BlockSpec((bs, D), lambda i: (i, 0))],
        out_specs=pl.BlockSpec((bs, D, N), lambda i: (i, 0, 0)),
        out_shape=jax.ShapeDtypeStruct((S, D, N), jnp.float32),
    )(rows)
    return jnp.transpose(out_t, (2, 0, 1))


# speedup vs baseline: 1.0186x; 1.0186x over previous
import jax
import jax.numpy as jnp
from jax import lax
from jax.experimental import pallas as pl
from jax.experimental.pallas import tpu as pltpu
from jax.experimental.pallas import tpu_sc as plsc

_BS = 8


_IDX_W = 112


def _sc_lookup_body(table_hbm, rows_hbm, idx_v, rows_v, sem):
    first = (lax.axis_index("c") == 0) & (lax.axis_index("s") == 0)

    @pl.when(first)
    def _():
        S = rows_hbm.shape[0]
        for j in range(2):
            for i in range(_IDX_W // 16):
                idx_v[j, pl.ds(i * 16, 16)] = lax.iota(jnp.int32, 16) + (
                    j * _IDX_W + i * 16
                )
        burst = [
            pltpu.async_copy(
                table_hbm.at[idx_v.at[j]],
                rows_v.at[pl.ds(j * _IDX_W, _IDX_W)],
                sem,
            )
            for j in range(2)
        ]
        for c in burst:
            c.wait()
        pltpu.sync_copy(rows_v.at[pl.ds(0, S)], rows_hbm)


def _tc_splat_body(rows_ref, out_ref):
    out_ref[...] = jnp.broadcast_to(
        rows_ref[...][:, :, None], out_ref.shape
    )


def kernel(x, pos_table):
    N, S = x.shape
    D = pos_table.shape[1]

    mesh = plsc.VectorSubcoreMesh(
        core_axis_name="c", subcore_axis_name="s", num_cores=1
    )
    sc_lookup = pl.kernel(
        _sc_lookup_body,
        out_type=jax.ShapeDtypeStruct((S, D), jnp.float32),
        mesh=mesh,
        scratch_types=[
            pltpu.VMEM((2, _IDX_W), jnp.int32),
            pltpu.VMEM((2 * _IDX_W, D), jnp.float32),
            pltpu.SemaphoreType.DMA,
        ],
        compiler_params=pltpu.CompilerParams(use_tc_tiling_on_sc=False),
    )
    rows = sc_lookup(pos_table)

    bs = _BS
    while S % bs:
        bs //= 2
    out_t = pl.pallas_call(
        _tc_splat_body,
        grid=(S // bs,),
        in_specs=[pl.BlockSpec((bs, D), lambda i: (i, 0))],
        out_specs=pl.BlockSpec((bs, D, N), lambda i: (i, 0, 0)),
        out_shape=jax.ShapeDtypeStruct((S, D, N), jnp.float32),
    )(rows)
    return jnp.transpose(out_t, (2, 0, 1))
